# baseline (speedup 1.0000x reference)
import functools

import jax
import jax.numpy as jnp
from jax import lax
from jax.experimental import pallas as pl
from jax.experimental.pallas import tpu as pltpu
from jax.experimental.pallas import tpu_sc as plsc

NUM_ROWS = 16384
NUM_COLS = 26
DIM = 32
B = NUM_ROWS * NUM_COLS
V = 1000000

_info = plsc.get_sparse_core_info()
NC = _info.num_cores
NS = _info.num_subcores
NW = NC * NS
L = _info.num_lanes

_mesh = plsc.VectorSubcoreMesh(core_axis_name="c", subcore_axis_name="s")

R = 1792
N_CHUNKS_D = V // R
TAIL = V - N_CHUNKS_D * R
TAIL_I0 = N_CHUNKS_D * R
MAX_T = -(-N_CHUNKS_D // NW)


@functools.partial(
    pl.kernel,
    mesh=_mesh,
    out_type=jax.ShapeDtypeStruct((V // 4, 128), jnp.float32),
    compiler_params=pltpu.CompilerParams(needs_layout_passes=False),
    scratch_types=[
        pltpu.VMEM((DIM, R), jnp.float32),
        pltpu.VMEM((R // 4, 128), jnp.float32),
        pltpu.VMEM((DIM, TAIL), jnp.float32),
        pltpu.VMEM((TAIL // 4, 128), jnp.float32),
    ],
)
def _detile(pt_hbm, out_hbm, in_v, out_v, tin_v, tout_v):
    wid = lax.axis_index("s") * NC + lax.axis_index("c")

    lane = lax.iota(jnp.int32, L)
    d_lo = lane
    d_hi = lane + L

    def shuffle(src_v, dst_v, n_rows):
        def row(j, _):
            for k in range(4):
                i_vec = jnp.full((L,), 4 * j + k, dtype=jnp.int32)
                lo = plsc.load_gather(src_v, [d_lo, i_vec])
                hi = plsc.load_gather(src_v, [d_hi, i_vec])
                dst_v[j, pl.ds(32 * k, L)] = lo
                dst_v[j, pl.ds(32 * k + L, L)] = hi
            return _
        lax.fori_loop(0, n_rows, row, 0)

    def chunk(c):
        i0 = c * R
        pltpu.sync_copy(pt_hbm.at[:, pl.ds(i0, R)], in_v)
        shuffle(in_v, out_v, R // 4)
        pltpu.sync_copy(out_v, out_hbm.at[pl.ds(c * (R // 4), R // 4)])

    for t in range(MAX_T):
        c = wid + t * NW

        @pl.when(c < N_CHUNKS_D)
        def _():
            chunk(c)

    @pl.when(wid == NW - 1)
    def _():
        pltpu.sync_copy(pt_hbm.at[:, pl.ds(TAIL_I0, TAIL)], tin_v)
        shuffle(tin_v, tout_v, TAIL // 4)
        pltpu.sync_copy(tout_v, out_hbm.at[pl.ds(TAIL_I0 // 4, TAIL // 4)])


B_PER_W = B // NW
CHUNK = 1024
N_CHUNKS = B_PER_W // CHUNK
NBUF = 2


@functools.partial(
    pl.kernel,
    mesh=_mesh,
    out_type=jax.ShapeDtypeStruct((B, DIM), jnp.float32),
    compiler_params=pltpu.CompilerParams(use_tc_tiling_on_sc=False),
    scratch_types=[
        pltpu.VMEM((B_PER_W,), jnp.int32),
        pltpu.VMEM((NBUF, CHUNK, DIM), jnp.float32),
        [pltpu.SemaphoreType.DMA] * NBUF,
        [pltpu.SemaphoreType.DMA] * NBUF,
    ],
)
def _gather(idx_hbm, table_hbm, out_hbm, idx_v, rows_v, gsems, ssems):
    wid = lax.axis_index("s") * NC + lax.axis_index("c")
    base = wid * B_PER_W

    pltpu.sync_copy(idx_hbm.at[pl.ds(base, B_PER_W)], idx_v)

    def start_gather(i):
        b = i % NBUF
        return pltpu.async_copy(
            table_hbm.at[idx_v.at[pl.ds(i * CHUNK, CHUNK)]],
            rows_v.at[b],
            gsems[b],
        )

    def start_store(i):
        b = i % NBUF
        return pltpu.async_copy(
            rows_v.at[b],
            out_hbm.at[pl.ds(base + i * CHUNK, CHUNK)],
            ssems[b],
        )

    gathers = [None] * N_CHUNKS
    stores = [None] * N_CHUNKS
    gathers[0] = start_gather(0)
    for i in range(N_CHUNKS):
        if i + 1 < N_CHUNKS:
            if i + 1 >= NBUF:
                stores[i + 1 - NBUF].wait()
            gathers[i + 1] = start_gather(i + 1)
        gathers[i].wait()
        stores[i] = start_store(i)
    for i in range(N_CHUNKS - NBUF, N_CHUNKS):
        stores[i].wait()


def kernel(input, embedding_weight):
    idx = input.reshape(-1).astype(jnp.int32)
    compact = _detile(embedding_weight.T)
    table = compact.reshape(V, DIM)
    out = _gather(idx, table)
    return out.reshape(NUM_ROWS, NUM_COLS, DIM)


# speedup vs baseline: 1.1992x; 1.1992x over previous
import functools

import jax
import jax.numpy as jnp
from jax import lax
from jax.experimental import pallas as pl
from jax.experimental.pallas import tpu as pltpu
from jax.experimental.pallas import tpu_sc as plsc

NUM_ROWS = 16384
NUM_COLS = 26
DIM = 32
B = NUM_ROWS * NUM_COLS
V = 1000000

_info = plsc.get_sparse_core_info()
NC = _info.num_cores
NS = _info.num_subcores
NW = NC * NS
L = _info.num_lanes

_mesh = plsc.VectorSubcoreMesh(core_axis_name="c", subcore_axis_name="s")

R = 1792
N_CHUNKS_D = V // R
TAIL = V - N_CHUNKS_D * R
TAIL_I0 = N_CHUNKS_D * R
MAX_T = -(-N_CHUNKS_D // NW)


@functools.partial(
    pl.kernel,
    mesh=_mesh,
    out_type=jax.ShapeDtypeStruct((V // 4, 128), jnp.float32),
    compiler_params=pltpu.CompilerParams(needs_layout_passes=False),
    scratch_types=[
        pltpu.VMEM((DIM, R), jnp.float32),
        pltpu.VMEM((R // 4, 128), jnp.float32),
        pltpu.VMEM((DIM, TAIL), jnp.float32),
        pltpu.VMEM((TAIL // 4, 128), jnp.float32),
    ],
)
def _detile(pt_hbm, out_hbm, in_v, out_v, tin_v, tout_v):
    wid = lax.axis_index("s") * NC + lax.axis_index("c")

    lane = lax.iota(jnp.int32, L)
    d_lo = lane
    d_hi = lane + L

    def shuffle(src_v, dst_v, n_rows):
        @plsc.parallel_loop(0, n_rows, unroll=8)
        def row(j):
            for k in range(4):
                i_vec = jnp.full((L,), 4 * j + k, dtype=jnp.int32)
                lo = plsc.load_gather(src_v, [d_lo, i_vec])
                hi = plsc.load_gather(src_v, [d_hi, i_vec])
                dst_v[j, pl.ds(32 * k, L)] = lo
                dst_v[j, pl.ds(32 * k + L, L)] = hi

    def chunk(c):
        i0 = c * R
        pltpu.sync_copy(pt_hbm.at[:, pl.ds(i0, R)], in_v)
        shuffle(in_v, out_v, R // 4)
        pltpu.sync_copy(out_v, out_hbm.at[pl.ds(c * (R // 4), R // 4)])

    for t in range(MAX_T):
        c = wid + t * NW

        @pl.when(c < N_CHUNKS_D)
        def _():
            chunk(c)

    @pl.when(wid == NW - 1)
    def _():
        pltpu.sync_copy(pt_hbm.at[:, pl.ds(TAIL_I0, TAIL)], tin_v)
        shuffle(tin_v, tout_v, TAIL // 4)
        pltpu.sync_copy(tout_v, out_hbm.at[pl.ds(TAIL_I0 // 4, TAIL // 4)])


B_PER_W = B // NW
CHUNK = 1024
N_CHUNKS = B_PER_W // CHUNK
NBUF = 2


@functools.partial(
    pl.kernel,
    mesh=_mesh,
    out_type=jax.ShapeDtypeStruct((B, DIM), jnp.float32),
    compiler_params=pltpu.CompilerParams(use_tc_tiling_on_sc=False),
    scratch_types=[
        pltpu.VMEM((B_PER_W,), jnp.int32),
        pltpu.VMEM((NBUF, CHUNK, DIM), jnp.float32),
        [pltpu.SemaphoreType.DMA] * NBUF,
        [pltpu.SemaphoreType.DMA] * NBUF,
    ],
)
def _gather(idx_hbm, table_hbm, out_hbm, idx_v, rows_v, gsems, ssems):
    wid = lax.axis_index("s") * NC + lax.axis_index("c")
    base = wid * B_PER_W

    pltpu.sync_copy(idx_hbm.at[pl.ds(base, B_PER_W)], idx_v)

    def start_gather(i):
        b = i % NBUF
        return pltpu.async_copy(
            table_hbm.at[idx_v.at[pl.ds(i * CHUNK, CHUNK)]],
            rows_v.at[b],
            gsems[b],
        )

    def start_store(i):
        b = i % NBUF
        return pltpu.async_copy(
            rows_v.at[b],
            out_hbm.at[pl.ds(base + i * CHUNK, CHUNK)],
            ssems[b],
        )

    gathers = [None] * N_CHUNKS
    stores = [None] * N_CHUNKS
    gathers[0] = start_gather(0)
    for i in range(N_CHUNKS):
        if i + 1 < N_CHUNKS:
            if i + 1 >= NBUF:
                stores[i + 1 - NBUF].wait()
            gathers[i + 1] = start_gather(i + 1)
        gathers[i].wait()
        stores[i] = start_store(i)
    for i in range(N_CHUNKS - NBUF, N_CHUNKS):
        stores[i].wait()


def kernel(input, embedding_weight):
    idx = input.reshape(-1).astype(jnp.int32)
    compact = _detile(embedding_weight.T)
    table = compact.reshape(V, DIM)
    out = _gather(idx, table)
    return out.reshape(NUM_ROWS, NUM_COLS, DIM)


# speedup vs baseline: 2.2249x; 1.8553x over previous
import functools

import jax
import jax.numpy as jnp
from jax import lax
from jax.experimental import pallas as pl
from jax.experimental.pallas import tpu as pltpu
from jax.experimental.pallas import tpu_sc as plsc

NUM_ROWS = 16384
NUM_COLS = 26
DIM = 32
B = NUM_ROWS * NUM_COLS
V = 1000000

_info = plsc.get_sparse_core_info()
NC = _info.num_cores
NS = _info.num_subcores
NW = NC * NS
L = _info.num_lanes

_mesh = plsc.VectorSubcoreMesh(core_axis_name="c", subcore_axis_name="s")

R = 1792
N_CHUNKS_D = V // R
TAIL = V - N_CHUNKS_D * R
TAIL_I0 = N_CHUNKS_D * R
MAX_T = -(-N_CHUNKS_D // NW)


@functools.partial(
    pl.kernel,
    mesh=_mesh,
    out_type=jax.ShapeDtypeStruct((V // 4, 128), jnp.float32),
    compiler_params=pltpu.CompilerParams(needs_layout_passes=False),
    scratch_types=[
        pltpu.VMEM((DIM, R), jnp.float32),
        pltpu.VMEM((R // 4, 128), jnp.float32),
        pltpu.VMEM((DIM, TAIL), jnp.float32),
        pltpu.VMEM((TAIL // 4, 128), jnp.float32),
    ],
)
def _detile(pt_hbm, out_hbm, in_v, out_v, tin_v, tout_v):
    wid = lax.axis_index("s") * NC + lax.axis_index("c")

    lane = lax.iota(jnp.int32, L)
    dvecs = [(lane + s) % L + L * g for g in range(2) for s in range(L)]

    def shuffle(src_v, dst_v, n_rows):
        @plsc.parallel_loop(0, n_rows // 4, unroll=2)
        def block(ib):
            i_vec = ib * L + lane
            j_vec = lax.shift_right_logical(i_vec, 2)
            k_vec = lax.shift_left(jnp.bitwise_and(i_vec, 3), 5)
            for dv in dvecs:
                v = plsc.load_gather(src_v, [dv, i_vec])
                plsc.store_scatter(dst_v, [j_vec, k_vec + dv], v)

    def chunk(c):
        i0 = c * R
        pltpu.sync_copy(pt_hbm.at[:, pl.ds(i0, R)], in_v)
        shuffle(in_v, out_v, R // 4)
        pltpu.sync_copy(out_v, out_hbm.at[pl.ds(c * (R // 4), R // 4)])

    def round_(t, carry):
        c = wid + t * NW

        @pl.when(c < N_CHUNKS_D)
        def _():
            chunk(c)

        return carry

    lax.fori_loop(0, MAX_T, round_, 0)

    @pl.when(wid == NW - 1)
    def _():
        pltpu.sync_copy(pt_hbm.at[:, pl.ds(TAIL_I0, TAIL)], tin_v)
        shuffle(tin_v, tout_v, TAIL // 4)
        pltpu.sync_copy(tout_v, out_hbm.at[pl.ds(TAIL_I0 // 4, TAIL // 4)])


B_PER_W = B // NW
CHUNK = 1024
N_CHUNKS = B_PER_W // CHUNK
NBUF = 2


@functools.partial(
    pl.kernel,
    mesh=_mesh,
    out_type=jax.ShapeDtypeStruct((B, DIM), jnp.float32),
    compiler_params=pltpu.CompilerParams(use_tc_tiling_on_sc=False),
    scratch_types=[
        pltpu.VMEM((B_PER_W,), jnp.int32),
        pltpu.VMEM((NBUF, CHUNK, DIM), jnp.float32),
        [pltpu.SemaphoreType.DMA] * NBUF,
        [pltpu.SemaphoreType.DMA] * NBUF,
    ],
)
def _gather(idx_hbm, table_hbm, out_hbm, idx_v, rows_v, gsems, ssems):
    wid = lax.axis_index("s") * NC + lax.axis_index("c")
    base = wid * B_PER_W

    pltpu.sync_copy(idx_hbm.at[pl.ds(base, B_PER_W)], idx_v)

    def start_gather(i):
        b = i % NBUF
        return pltpu.async_copy(
            table_hbm.at[idx_v.at[pl.ds(i * CHUNK, CHUNK)]],
            rows_v.at[b],
            gsems[b],
        )

    def start_store(i):
        b = i % NBUF
        return pltpu.async_copy(
            rows_v.at[b],
            out_hbm.at[pl.ds(base + i * CHUNK, CHUNK)],
            ssems[b],
        )

    gathers = [None] * N_CHUNKS
    stores = [None] * N_CHUNKS
    gathers[0] = start_gather(0)
    for i in range(N_CHUNKS):
        if i + 1 < N_CHUNKS:
            if i + 1 >= NBUF:
                stores[i + 1 - NBUF].wait()
            gathers[i + 1] = start_gather(i + 1)
        gathers[i].wait()
        stores[i] = start_store(i)
    for i in range(N_CHUNKS - NBUF, N_CHUNKS):
        stores[i].wait()


def kernel(input, embedding_weight):
    idx = input.reshape(-1).astype(jnp.int32)
    compact = _detile(embedding_weight.T)
    table = compact.reshape(V, DIM)
    out = _gather(idx, table)
    return out.reshape(NUM_ROWS, NUM_COLS, DIM)


# speedup vs baseline: 2.9512x; 1.3264x over previous
import functools

import jax
import jax.numpy as jnp
from jax import lax
from jax.experimental import pallas as pl
from jax.experimental.pallas import tpu as pltpu
from jax.experimental.pallas import tpu_sc as plsc

NUM_ROWS = 16384
NUM_COLS = 26
DIM = 32
B = NUM_ROWS * NUM_COLS
V = 1000000

_info = plsc.get_sparse_core_info()
NC = _info.num_cores
NS = _info.num_subcores
NW = NC * NS
L = _info.num_lanes

_mesh = plsc.VectorSubcoreMesh(core_axis_name="c", subcore_axis_name="s")

R = 1792
N_CHUNKS_D = V // R
TAIL = V - N_CHUNKS_D * R
TAIL_I0 = N_CHUNKS_D * R
MAX_T = -(-N_CHUNKS_D // NW)


@functools.partial(
    pl.kernel,
    mesh=_mesh,
    out_type=jax.ShapeDtypeStruct((V // 4, 128), jnp.float32),
    compiler_params=pltpu.CompilerParams(needs_layout_passes=False),
    scratch_types=[
        pltpu.VMEM((DIM, R), jnp.float32),
        pltpu.VMEM((R // 4, 128), jnp.float32),
        pltpu.VMEM((DIM, TAIL), jnp.float32),
        pltpu.VMEM((TAIL // 4, 128), jnp.float32),
    ],
)
def _detile(pt_hbm, out_hbm, in_v, out_v, tin_v, tout_v):
    wid = lax.axis_index("s") * NC + lax.axis_index("c")

    lane = lax.iota(jnp.int32, L)
    dvecs = [(lane + s) % L + L * g for g in range(2) for s in range(L)]

    def shuffle(src_v, dst_v, n_rows):
        @plsc.parallel_loop(0, n_rows // 4, unroll=2)
        def block(ib):
            i_vec = ib * L + lane
            j_vec = lax.shift_right_logical(i_vec, 2)
            k_vec = lax.shift_left(jnp.bitwise_and(i_vec, 3), 5)
            for dv in dvecs:
                v = plsc.load_gather(src_v, [dv, i_vec])
                plsc.store_scatter(dst_v, [j_vec, k_vec + dv], v)

    def chunk(c):
        i0 = c * R
        pltpu.sync_copy(pt_hbm.at[:, pl.ds(i0, R)], in_v)
        shuffle(in_v, out_v, R // 4)
        pltpu.sync_copy(out_v, out_hbm.at[pl.ds(c * (R // 4), R // 4)])

    def round_(t, carry):
        c = wid + t * NW

        @pl.when(c < N_CHUNKS_D)
        def _():
            chunk(c)

        return carry

    lax.fori_loop(0, MAX_T, round_, 0)

    @pl.when(wid == NW - 1)
    def _():
        pltpu.sync_copy(pt_hbm.at[:, pl.ds(TAIL_I0, TAIL)], tin_v)
        shuffle(tin_v, tout_v, TAIL // 4)
        pltpu.sync_copy(tout_v, out_hbm.at[pl.ds(TAIL_I0 // 4, TAIL // 4)])


B_PER_W = B // NW
CHUNK = 1024
N_CHUNKS = B_PER_W // CHUNK
NBUF = 2


@functools.partial(
    pl.kernel,
    mesh=_mesh,
    out_type=jax.ShapeDtypeStruct((B, DIM), jnp.float32),
    compiler_params=pltpu.CompilerParams(use_tc_tiling_on_sc=False),
    scratch_types=[
        pltpu.VMEM((B_PER_W,), jnp.int32),
        pltpu.VMEM((NBUF, CHUNK, DIM), jnp.float32),
        [pltpu.SemaphoreType.DMA] * NBUF,
        [pltpu.SemaphoreType.DMA] * NBUF,
    ],
)
def _gather(idx_hbm, table_hbm, out_hbm, idx_v, rows_v, gsems, ssems):
    wid = lax.axis_index("s") * NC + lax.axis_index("c")
    base = wid * B_PER_W

    pltpu.sync_copy(idx_hbm.at[pl.ds(base, B_PER_W)], idx_v)

    def start_gather(i):
        b = i % NBUF
        return pltpu.async_copy(
            table_hbm.at[idx_v.at[pl.ds(i * CHUNK, CHUNK)]],
            rows_v.at[b],
            gsems[b],
        )

    def start_store(i):
        b = i % NBUF
        return pltpu.async_copy(
            rows_v.at[b],
            out_hbm.at[pl.ds(base + i * CHUNK, CHUNK)],
            ssems[b],
        )

    gathers = [None] * N_CHUNKS
    stores = [None] * N_CHUNKS
    gathers[0] = start_gather(0)
    for i in range(N_CHUNKS):
        if i + 1 < N_CHUNKS:
            if i + 1 >= NBUF:
                stores[i + 1 - NBUF].wait()
            gathers[i + 1] = start_gather(i + 1)
        gathers[i].wait()
        stores[i] = start_store(i)
    for i in range(N_CHUNKS - NBUF, N_CHUNKS):
        stores[i].wait()


ROW = NUM_COLS * DIM
BC = NUM_ROWS // 128
BC_PER_W = BC // NW
CH = NUM_COLS // 2


@functools.partial(
    pl.kernel,
    mesh=_mesh,
    out_type=jax.ShapeDtypeStruct((NUM_COLS, 4, BC, 8, 128), jnp.float32),
    compiler_params=pltpu.CompilerParams(needs_layout_passes=False),
    scratch_types=[
        pltpu.VMEM((64 * ROW,), jnp.float32),
        pltpu.VMEM((CH, 4, 8, 128), jnp.float32),
    ],
)
def _retile(flat_hbm, out_hbm, in_v, out_v):
    wid = lax.axis_index("s") * NC + lax.axis_index("c")

    lane = lax.iota(jnp.int32, L)
    lane_row = lane * ROW
    diags = [(lane + s) % 8 for s in range(8)]

    def do_bc(bcw, carry):
        bc = wid * BC_PER_W + bcw
        for ch in range(2):
            for h in range(2):
                pltpu.sync_copy(
                    flat_hbm.at[pl.ds((bc * 128 + 64 * h) * ROW, 64 * ROW)],
                    in_v,
                )

                @plsc.parallel_loop(0, CH * 4, unroll=1)
                def fill(ct):
                    c = ct // 4
                    tr = ct % 4
                    sbase = 32 * (ch * CH + c) + 8 * tr
                    c_vec = jnp.full((L,), c, dtype=jnp.int32)
                    t_vec = jnp.full((L,), tr, dtype=jnp.int32)
                    for q in range(4):
                        abase = lane_row + (q * 16 * ROW + sbase)
                        lbase = lane + (64 * h + q * 16)
                        for dv in diags:
                            v = plsc.load_gather(in_v, [abase + dv])
                            plsc.store_scatter(
                                out_v, [c_vec, t_vec, dv, lbase], v
                            )

            for c in range(CH):
                for tr in range(4):
                    pltpu.sync_copy(
                        out_v.at[c, tr],
                        out_hbm.at[ch * CH + c, tr, bc],
                    )
        return carry

    lax.fori_loop(0, BC_PER_W, do_bc, 0)


def kernel(input, embedding_weight):
    idx = input.reshape(-1).astype(jnp.int32)
    compact = _detile(embedding_weight.T)
    table = compact.reshape(V, DIM)
    out = _gather(idx, table)
    out5 = _retile(out.reshape(-1))
    return out5.transpose(2, 4, 0, 1, 3).reshape(NUM_ROWS, NUM_COLS, DIM)


# speedup vs baseline: 3.1417x; 1.0645x over previous
import functools

import jax
import jax.numpy as jnp
from jax import lax
from jax.experimental import pallas as pl
from jax.experimental.pallas import tpu as pltpu
from jax.experimental.pallas import tpu_sc as plsc

NUM_ROWS = 16384
NUM_COLS = 26
DIM = 32
B = NUM_ROWS * NUM_COLS
V = 1000000

_info = plsc.get_sparse_core_info()
NC = _info.num_cores
NS = _info.num_subcores
NW = NC * NS
L = _info.num_lanes

_mesh = plsc.VectorSubcoreMesh(core_axis_name="c", subcore_axis_name="s")

R = 1792
N_CHUNKS_D = V // R
TAIL = V - N_CHUNKS_D * R
TAIL_I0 = N_CHUNKS_D * R
MAX_T = -(-N_CHUNKS_D // NW)


@functools.partial(
    pl.kernel,
    mesh=_mesh,
    out_type=jax.ShapeDtypeStruct((V // 4, 128), jnp.float32),
    compiler_params=pltpu.CompilerParams(needs_layout_passes=False),
    scratch_types=[
        pltpu.VMEM((DIM, R), jnp.float32),
        pltpu.VMEM((R // 4, 128), jnp.float32),
        pltpu.VMEM((DIM, TAIL), jnp.float32),
        pltpu.VMEM((TAIL // 4, 128), jnp.float32),
    ],
)
def _detile(pt_hbm, out_hbm, in_v, out_v, tin_v, tout_v):
    wid = lax.axis_index("s") * NC + lax.axis_index("c")

    lane = lax.iota(jnp.int32, L)
    dvecs = [(lane + s) % L + L * g for g in range(2) for s in range(L)]

    def shuffle(src_v, dst_v, n_rows):
        @plsc.parallel_loop(0, n_rows // 4, unroll=2)
        def block(ib):
            i_vec = ib * L + lane
            j_vec = lax.shift_right_logical(i_vec, 2)
            k_vec = lax.shift_left(jnp.bitwise_and(i_vec, 3), 5)
            for dv in dvecs:
                v = plsc.load_gather(src_v, [dv, i_vec])
                plsc.store_scatter(dst_v, [j_vec, k_vec + dv], v)

    def chunk(c):
        i0 = c * R
        pltpu.sync_copy(pt_hbm.at[:, pl.ds(i0, R)], in_v)
        shuffle(in_v, out_v, R // 4)
        pltpu.sync_copy(out_v, out_hbm.at[pl.ds(c * (R // 4), R // 4)])

    def round_(t, carry):
        c = wid + t * NW

        @pl.when(c < N_CHUNKS_D)
        def _():
            chunk(c)

        return carry

    lax.fori_loop(0, MAX_T, round_, 0)

    @pl.when(wid == NW - 1)
    def _():
        pltpu.sync_copy(pt_hbm.at[:, pl.ds(TAIL_I0, TAIL)], tin_v)
        shuffle(tin_v, tout_v, TAIL // 4)
        pltpu.sync_copy(tout_v, out_hbm.at[pl.ds(TAIL_I0 // 4, TAIL // 4)])


B_PER_W = B // NW
CHUNK = 1024
N_CHUNKS = B_PER_W // CHUNK
NBUF = 2


@functools.partial(
    pl.kernel,
    mesh=_mesh,
    out_type=jax.ShapeDtypeStruct((B, DIM), jnp.float32),
    compiler_params=pltpu.CompilerParams(use_tc_tiling_on_sc=False),
    scratch_types=[
        pltpu.VMEM((B_PER_W,), jnp.int32),
        pltpu.VMEM((NBUF, CHUNK, DIM), jnp.float32),
        [pltpu.SemaphoreType.DMA] * NBUF,
        [pltpu.SemaphoreType.DMA] * NBUF,
    ],
)
def _gather(idx_hbm, table_hbm, out_hbm, idx_v, rows_v, gsems, ssems):
    wid = lax.axis_index("s") * NC + lax.axis_index("c")
    base = wid * B_PER_W

    pltpu.sync_copy(idx_hbm.at[pl.ds(base, B_PER_W)], idx_v)

    def start_gather(i):
        b = i % NBUF
        return pltpu.async_copy(
            table_hbm.at[idx_v.at[pl.ds(i * CHUNK, CHUNK)]],
            rows_v.at[b],
            gsems[b],
        )

    def start_store(i):
        b = i % NBUF
        return pltpu.async_copy(
            rows_v.at[b],
            out_hbm.at[pl.ds(base + i * CHUNK, CHUNK)],
            ssems[b],
        )

    gathers = [None] * N_CHUNKS
    stores = [None] * N_CHUNKS
    gathers[0] = start_gather(0)
    for i in range(N_CHUNKS):
        if i + 1 < N_CHUNKS:
            if i + 1 >= NBUF:
                stores[i + 1 - NBUF].wait()
            gathers[i + 1] = start_gather(i + 1)
        gathers[i].wait()
        stores[i] = start_store(i)
    for i in range(N_CHUNKS - NBUF, N_CHUNKS):
        stores[i].wait()


BC = NUM_ROWS // 128
BC_PER_W = BC // NW
CH = NUM_COLS // 2
BLK = 128 * DIM


@functools.partial(
    pl.kernel,
    mesh=_mesh,
    out_type=jax.ShapeDtypeStruct((NUM_COLS, 4, BC, 8, 128), jnp.float32),
    compiler_params=pltpu.CompilerParams(needs_layout_passes=False),
    scratch_types=[
        pltpu.VMEM((CH * BLK,), jnp.float32),
        pltpu.VMEM((CH, 4, 8, 128), jnp.float32),
        pltpu.SemaphoreType.DMA,
    ],
)
def _retile(flat_hbm, out_hbm, in_v, out_v, sem):
    wid = lax.axis_index("s") * NC + lax.axis_index("c")

    lane = lax.iota(jnp.int32, L)
    lane32 = lane * DIM
    diags = [(lane + s) % 8 for s in range(8)]

    def do_bc(bcw, carry):
        bc = wid * BC_PER_W + bcw
        for ch in range(2):
            slabs = [
                pltpu.async_copy(
                    flat_hbm.at[
                        pl.ds(((ch * CH + cl) * NUM_ROWS + 128 * bc) * DIM, BLK)
                    ],
                    in_v.at[pl.ds(cl * BLK, BLK)],
                    sem,
                )
                for cl in range(CH)
            ]
            for s_ in slabs:
                s_.wait()

            @plsc.parallel_loop(0, CH * 4, unroll=1)
            def fill(ct):
                c = ct // 4
                tr = ct % 4
                c_vec = jnp.full((L,), c, dtype=jnp.int32)
                t_vec = jnp.full((L,), tr, dtype=jnp.int32)
                for q in range(8):
                    pbase = lane32 + c * BLK + (q * 16 * DIM + 8 * tr)
                    lbase = lane + q * 16
                    for dv in diags:
                        v = plsc.load_gather(in_v, [pbase + dv])
                        plsc.store_scatter(
                            out_v, [c_vec, t_vec, dv, lbase], v
                        )

            pltpu.sync_copy(out_v, out_hbm.at[pl.ds(ch * CH, CH), :, bc])
        return carry

    lax.fori_loop(0, BC_PER_W, do_bc, 0)


def kernel(input, embedding_weight):
    idx = input.T.reshape(-1).astype(jnp.int32)
    compact = _detile(embedding_weight.T)
    table = compact.reshape(V, DIM)
    out = _gather(idx, table)
    out5 = _retile(out.reshape(-1))
    return out5.transpose(2, 4, 0, 1, 3).reshape(NUM_ROWS, NUM_COLS, DIM)


# speedup vs baseline: 3.3510x; 1.0666x over previous
import functools

import jax
import jax.numpy as jnp
from jax import lax
from jax.experimental import pallas as pl
from jax.experimental.pallas import tpu as pltpu
from jax.experimental.pallas import tpu_sc as plsc

NUM_ROWS = 16384
NUM_COLS = 26
DIM = 32
B = NUM_ROWS * NUM_COLS
V = 1000000

_info = plsc.get_sparse_core_info()
NC = _info.num_cores
NS = _info.num_subcores
NW = NC * NS
L = _info.num_lanes

_mesh = plsc.VectorSubcoreMesh(core_axis_name="c", subcore_axis_name="s")

R = 896
N_CHUNKS_D = V // R
TAIL = V - N_CHUNKS_D * R
TAIL_I0 = N_CHUNKS_D * R
MAX_T = 36


@functools.partial(
    pl.kernel,
    mesh=_mesh,
    out_type=jax.ShapeDtypeStruct((V // 4, 128), jnp.float32),
    compiler_params=pltpu.CompilerParams(needs_layout_passes=False),
    scratch_types=[
        pltpu.VMEM((2, DIM, R), jnp.float32),
        pltpu.VMEM((2, R // 4, 128), jnp.float32),
        pltpu.VMEM((DIM, TAIL), jnp.float32),
        pltpu.VMEM((TAIL // 4, 128), jnp.float32),
        [pltpu.SemaphoreType.DMA] * 2,
        [pltpu.SemaphoreType.DMA] * 2,
    ],
)
def _detile(pt_hbm, out_hbm, in_v, out_v, tin_v, tout_v, lsems, ssems):
    wid = lax.axis_index("s") * NC + lax.axis_index("c")

    lane = lax.iota(jnp.int32, L)
    dvecs = [(lane + s) % L + L * g for g in range(2) for s in range(L)]

    def shuffle(src_v, dst_v, n_rows):
        @plsc.parallel_loop(0, n_rows // 4, unroll=2)
        def block(ib):
            i_vec = ib * L + lane
            j_vec = lax.shift_right_logical(i_vec, 2)
            k_vec = lax.shift_left(jnp.bitwise_and(i_vec, 3), 5)
            for dv in dvecs:
                v = plsc.load_gather(src_v, [dv, i_vec])
                plsc.store_scatter(dst_v, [j_vec, k_vec + dv], v)

    def start_load(c, b):
        return pltpu.async_copy(
            pt_hbm.at[:, pl.ds(c * R, R)], in_v.at[b], lsems[b]
        )

    def start_store(c, b):
        return pltpu.async_copy(
            out_v.at[b], out_hbm.at[pl.ds(c * (R // 4), R // 4)], ssems[b]
        )

    last = N_CHUNKS_D - 1

    def round_(t, carry):
        c0 = jnp.minimum(wid + (2 * t) * NW, last)
        c1 = jnp.minimum(wid + (2 * t + 1) * NW, last)
        l0 = start_load(c0, 0)
        l1 = start_load(c1, 1)
        l0.wait()
        shuffle(in_v.at[0], out_v.at[0], R // 4)
        s0 = start_store(c0, 0)
        l1.wait()
        shuffle(in_v.at[1], out_v.at[1], R // 4)
        s1 = start_store(c1, 1)
        s0.wait()
        s1.wait()
        return carry

    lax.fori_loop(0, MAX_T // 2, round_, 0)

    @pl.when(wid == NW - 1)
    def _():
        pltpu.sync_copy(pt_hbm.at[:, pl.ds(TAIL_I0, TAIL)], tin_v)
        shuffle(tin_v, tout_v, TAIL // 4)
        pltpu.sync_copy(tout_v, out_hbm.at[pl.ds(TAIL_I0 // 4, TAIL // 4)])


B_PER_W = B // NW
CHUNK = 1024
N_CHUNKS = B_PER_W // CHUNK
NBUF = 2


@functools.partial(
    pl.kernel,
    mesh=_mesh,
    out_type=jax.ShapeDtypeStruct((B, DIM), jnp.float32),
    compiler_params=pltpu.CompilerParams(use_tc_tiling_on_sc=False),
    scratch_types=[
        pltpu.VMEM((B_PER_W,), jnp.int32),
        pltpu.VMEM((NBUF, CHUNK, DIM), jnp.float32),
        [pltpu.SemaphoreType.DMA] * NBUF,
        [pltpu.SemaphoreType.DMA] * NBUF,
    ],
)
def _gather(idx_hbm, table_hbm, out_hbm, idx_v, rows_v, gsems, ssems):
    wid = lax.axis_index("s") * NC + lax.axis_index("c")
    base = wid * B_PER_W

    pltpu.sync_copy(idx_hbm.at[pl.ds(base, B_PER_W)], idx_v)

    def start_gather(i):
        b = i % NBUF
        return pltpu.async_copy(
            table_hbm.at[idx_v.at[pl.ds(i * CHUNK, CHUNK)]],
            rows_v.at[b],
            gsems[b],
        )

    def start_store(i):
        b = i % NBUF
        return pltpu.async_copy(
            rows_v.at[b],
            out_hbm.at[pl.ds(base + i * CHUNK, CHUNK)],
            ssems[b],
        )

    gathers = [None] * N_CHUNKS
    stores = [None] * N_CHUNKS
    gathers[0] = start_gather(0)
    for i in range(N_CHUNKS):
        if i + 1 < N_CHUNKS:
            if i + 1 >= NBUF:
                stores[i + 1 - NBUF].wait()
            gathers[i + 1] = start_gather(i + 1)
        gathers[i].wait()
        stores[i] = start_store(i)
    for i in range(N_CHUNKS - NBUF, N_CHUNKS):
        stores[i].wait()


BC = NUM_ROWS // 128
BC_PER_W = BC // NW
CH = NUM_COLS // 2
BLK = 128 * DIM


@functools.partial(
    pl.kernel,
    mesh=_mesh,
    out_type=jax.ShapeDtypeStruct((NUM_COLS, 4, BC, 8, 128), jnp.float32),
    compiler_params=pltpu.CompilerParams(needs_layout_passes=False),
    scratch_types=[
        pltpu.VMEM((CH * BLK,), jnp.float32),
        pltpu.VMEM((CH, 4, 8, 128), jnp.float32),
        pltpu.SemaphoreType.DMA,
    ],
)
def _retile(flat_hbm, out_hbm, in_v, out_v, sem):
    wid = lax.axis_index("s") * NC + lax.axis_index("c")

    lane = lax.iota(jnp.int32, L)
    lane32 = lane * DIM
    diags = [(lane + s) % 8 for s in range(8)]

    def do_bc(bcw, carry):
        bc = wid * BC_PER_W + bcw
        for ch in range(2):
            slabs = [
                pltpu.async_copy(
                    flat_hbm.at[
                        pl.ds(((ch * CH + cl) * NUM_ROWS + 128 * bc) * DIM, BLK)
                    ],
                    in_v.at[pl.ds(cl * BLK, BLK)],
                    sem,
                )
                for cl in range(CH)
            ]
            for s_ in slabs:
                s_.wait()

            @plsc.parallel_loop(0, CH * 4, unroll=1)
            def fill(ct):
                c = ct // 4
                tr = ct % 4
                c_vec = jnp.full((L,), c, dtype=jnp.int32)
                t_vec = jnp.full((L,), tr, dtype=jnp.int32)
                for q in range(8):
                    pbase = lane32 + c * BLK + (q * 16 * DIM + 8 * tr)
                    lbase = lane + q * 16
                    for dv in diags:
                        v = plsc.load_gather(in_v, [pbase + dv])
                        plsc.store_scatter(
                            out_v, [c_vec, t_vec, dv, lbase], v
                        )

            pltpu.sync_copy(out_v, out_hbm.at[pl.ds(ch * CH, CH), :, bc])
        return carry

    lax.fori_loop(0, BC_PER_W, do_bc, 0)


def kernel(input, embedding_weight):
    idx = input.T.reshape(-1).astype(jnp.int32)
    compact = _detile(embedding_weight.T)
    table = compact.reshape(V, DIM)
    out = _gather(idx, table)
    out5 = _retile(out.reshape(-1))
    return out5.transpose(2, 4, 0, 1, 3).reshape(NUM_ROWS, NUM_COLS, DIM)
